# initial kernel scaffold (unmeasured)
import jax
import jax.numpy as jnp
from jax import lax
from jax.experimental import pallas as pl
from jax.experimental.pallas import tpu as pltpu


def kernel(
    x,
):
    def body(*refs):
        pass

    out_shape = jax.ShapeDtypeStruct(..., jnp.float32)
    return pl.pallas_call(body, out_shape=out_shape)(...)


# baseline (device time: 1023367 ns/iter reference)
import jax
import jax.numpy as jnp
from jax import lax
from jax.experimental import pallas as pl
from jax.experimental.pallas import tpu as pltpu

N_DEV = 4
M_PER = 2048
M_GLOB = N_DEV * M_PER
N_COLS = 512
CDT = jnp.bfloat16


def _ce(v, j, k, flip):
    rows = v.shape[0]
    i = lax.broadcasted_iota(jnp.int32, (rows, 1), 0)
    is_lo = (i & j) == 0
    asc = jnp.logical_xor((i & k) == 0, flip)
    up = pltpu.roll(v, rows - j, 0)
    dn = pltpu.roll(v, j, 0)
    partner = jnp.where(is_lo, up, dn)
    lo = jnp.minimum(v, partner)
    hi = jnp.maximum(v, partner)
    return jnp.where(is_lo == asc, lo, hi)


def _sort_ref(ref, n_log, flip):

    def outer(k_log, _):
        k = jnp.int32(1) << k_log

        def inner(s, _):
            j = k >> (s + 1)
            ref[:, :] = _ce(ref[:, :], j, k, flip)
            return 0

        return lax.fori_loop(0, k_log, inner, 0)

    lax.fori_loop(1, n_log + 1, outer, 0)


def _merge_ref(ref, k_log_lo, k_log_hi):

    def outer(k_log, _):
        k = jnp.int32(1) << k_log

        def inner(s, _):
            j = k >> (s + 1)
            ref[:, :] = _ce(ref[:, :], j, k, jnp.bool_(False))
            return 0

        return lax.fori_loop(0, k_log, inner, 0)

    lax.fori_loop(k_log_lo, k_log_hi + 1, outer, 0)


def kernel(x):
    def body(x_ref, out_ref, local_ref, gather_ref, send_sems, recv_sems):
        my = lax.axis_index("i")
        right = lax.rem(my + 1, N_DEV)

        flip = lax.rem(my, 2) == 1
        local_ref[:, :] = x_ref[:, :].astype(CDT)
        _sort_ref(local_ref, 11, flip)
        gather_ref[pl.ds(my * M_PER, M_PER), :] = local_ref[:, :]

        for h in range(N_DEV - 1):
            so = lax.rem(my - h + N_DEV, N_DEV)
            rdma = pltpu.make_async_remote_copy(
                src_ref=gather_ref.at[pl.ds(so * M_PER, M_PER)],
                dst_ref=gather_ref.at[pl.ds(so * M_PER, M_PER)],
                send_sem=send_sems.at[h],
                recv_sem=recv_sems.at[h],
                device_id=(right,),
                device_id_type=pl.DeviceIdType.MESH,
            )
            rdma.start()
            rdma.wait()

        _merge_ref(gather_ref, 12, 13)

        out_ref[:, :] = gather_ref[pl.ds(my * M_PER, M_PER), :]

    return pl.pallas_call(
        body,
        out_shape=jax.ShapeDtypeStruct((M_PER, N_COLS), CDT),
        in_specs=[pl.BlockSpec(memory_space=pltpu.VMEM)],
        out_specs=pl.BlockSpec(memory_space=pltpu.VMEM),
        scratch_shapes=[
            pltpu.VMEM((M_PER, N_COLS), CDT),
            pltpu.VMEM((M_GLOB, N_COLS), CDT),
            pltpu.SemaphoreType.DMA((N_DEV - 1,)),
            pltpu.SemaphoreType.DMA((N_DEV - 1,)),
        ],
        compiler_params=pltpu.CompilerParams(
            vmem_limit_bytes=100 * 1024 * 1024,
        ),
    )(x)
